# speedup vs baseline: 10.0825x; 10.0825x over previous
import jax
import jax.numpy as jnp
from jax.experimental import pallas as pl
from jax.experimental.pallas import tpu as pltpu

BSIZE = 32


def _tile_kernel(mask_ref, x_ref, w_ref, b_ref, o_ref):
    n = pl.program_id(0)
    i = pl.program_id(1)
    j = pl.program_id(2)
    nbi = pl.num_programs(1)
    nbj = pl.num_programs(2)
    m = mask_ref[(n * nbi + i) * nbj + j]

    @pl.when(m != 0)
    def _active():
        c = x_ref.shape[3]
        oc = o_ref.shape[3]
        xb = x_ref[...].reshape(BSIZE * BSIZE, c)
        q = jnp.dot(xb, w_ref[...], preferred_element_type=jnp.float32)
        q = q + b_ref[...]
        o_ref[...] = q.reshape(1, BSIZE, BSIZE, oc)

    @pl.when(m == 0)
    def _inactive():
        o_ref[...] = jnp.zeros_like(o_ref)


def kernel(sbnet_x, active_block_indices, num_active, Wc, bc):
    n_batch, h, w, c = sbnet_x.shape
    oc = Wc.shape[-1]
    nbi = h // BSIZE
    nbj = w // BSIZE
    nblocks = n_batch * nbi * nbj

    idx = active_block_indices
    valid = jnp.arange(idx.shape[0]) < num_active
    flat = (idx[:, 0] * nbi + idx[:, 1]) * nbj + idx[:, 2]
    flat = jnp.where(valid, flat, nblocks)
    mask = jnp.zeros((nblocks,), dtype=jnp.int32).at[flat].set(
        1, mode="drop", unique_indices=True
    )

    w2 = Wc.reshape(c, oc)
    b2 = bc.reshape(1, oc)

    out = pl.pallas_call(
        _tile_kernel,
        grid_spec=pltpu.PrefetchScalarGridSpec(
            num_scalar_prefetch=1,
            grid=(n_batch, nbi, nbj),
            in_specs=[
                pl.BlockSpec((1, BSIZE, BSIZE, c), lambda n, i, j, m: (n, i, j, 0)),
                pl.BlockSpec((c, oc), lambda n, i, j, m: (0, 0)),
                pl.BlockSpec((1, oc), lambda n, i, j, m: (0, 0)),
            ],
            out_specs=pl.BlockSpec(
                (1, BSIZE, BSIZE, oc), lambda n, i, j, m: (n, i, j, 0)
            ),
        ),
        out_shape=jax.ShapeDtypeStruct((n_batch, nbi * BSIZE, nbj * BSIZE, oc),
                                       sbnet_x.dtype),
    )(mask, sbnet_x, w2, b2)
    return out
